# baseline (speedup 1.0000x reference)
import functools

import jax
import jax.numpy as jnp
from jax import lax
from jax.experimental import pallas as pl
from jax.experimental.pallas import tpu as pltpu
from jax.experimental.pallas import tpu_sc as plsc

F = 26
V = 100000
D = 16
B = 16384
N = B * F
G16 = V // D

_INFO = plsc.get_sparse_core_info()
NC = _INFO.num_cores
NS = _INFO.num_subcores
NW = NC * NS
CH = 64
NBUF = 4
LOOK = 2


def _sc_gather(granules, flat_idx, nf):
    n = B * nf
    rpw = n // NW
    nch = rpw // CH
    nout = nch // NBUF
    assert rpw * NW == n and nch * CH == rpw and rpw % nf == 0
    assert nch % NBUF == 0 and nout >= 3 and NBUF == LOOK + 2
    mesh = plsc.VectorSubcoreMesh(core_axis_name="c", subcore_axis_name="s")

    @functools.partial(
        pl.kernel,
        mesh=mesh,
        out_type=jax.ShapeDtypeStruct((n, D), jnp.float32),
        scratch_types=[
            pltpu.VMEM((rpw,), jnp.int32),
            pltpu.VMEM((NBUF, D, CH), jnp.int32),
            pltpu.VMEM((NBUF, D, CH, D), jnp.float32),
            pltpu.VMEM((NBUF, CH, D), jnp.float32),
        ] + [pltpu.SemaphoreType.DMA] * (2 * NBUF),
        compiler_params=pltpu.CompilerParams(
            use_tc_tiling_on_sc=False, needs_layout_passes=False
        ),
    )
    def k(tab_hbm, idx_hbm, out_hbm, idx_v, pring, gbufs, obufs, *sems):
        gsems, osems = sems[:NBUF], sems[NBUF:]
        wid = lax.axis_index("s") * NC + lax.axis_index("c")
        base = wid * rpw
        pltpu.sync_copy(idx_hbm.at[pl.ds(base, rpw)], idx_v)

        lanes = lax.iota(jnp.int32, 16)

        def prep_chunk(c, b):
            def grp(gi, carry):
                s = gi * 16
                p = c * CH + s
                v = idx_v[pl.ds(p, 16)]
                m0 = ((p + lanes) % nf) * V + (v >> 4)
                for d in range(D):
                    pring.at[b, d][pl.ds(s, 16)] = m0 + d * G16
                return carry

            lax.fori_loop(0, CH // 16, grp, 0)

        def gather_start(b):
            for d in range(D):
                pltpu.async_copy(
                    tab_hbm.at[pring.at[b, d]], gbufs.at[b, d], gsems[b]
                )

        def gather_wait(b):
            for d in range(D):
                pltpu.make_async_copy(
                    tab_hbm.at[pring.at[b, d]], gbufs.at[b, d], gsems[b]
                ).wait()

        def extract_chunk(c, b):
            ob = obufs.at[b]

            def grp(gi, carry):
                s = gi * 16
                v = idx_v[pl.ds(c * CH + s, 16)]
                lo = v & 15
                rows = s + lanes
                for d in range(D):
                    vals = plsc.load_gather(gbufs.at[b, d], [rows, lo])
                    plsc.store_scatter(ob, [rows, lanes * 0 + d], vals)
                return carry

            lax.fori_loop(0, CH // 16, grp, 0)

        def out_start(c, b):
            pltpu.async_copy(
                obufs.at[b], out_hbm.at[pl.ds(base + c * CH, CH)], osems[b]
            )

        def out_wait(c, b):
            pltpu.make_async_copy(
                obufs.at[b], out_hbm.at[pl.ds(base + c * CH, CH)], osems[b]
            ).wait()

        def step(c, b, with_owait, with_issue):
            gather_wait(b)
            extract_chunk(c, b)
            out_start(c, b)
            if with_issue:
                j = c + LOOK
                bj = (b + LOOK) % NBUF
                prep_chunk(j, bj)
                if with_owait:
                    out_wait(j - NBUF, bj)
                gather_start(bj)

        for j in range(LOOK):
            prep_chunk(j, j % NBUF)
            gather_start(j % NBUF)

        for b in range(NBUF):
            step(b, b, with_owait=(b >= NBUF - LOOK), with_issue=True)

        def outer(cb, carry):
            for b in range(NBUF):
                step(cb * NBUF + b, b, with_owait=True, with_issue=True)
            return carry

        lax.fori_loop(1, nout - 1, outer, 0)

        for b in range(NBUF):
            c = (nout - 1) * NBUF + b
            step(c, b, with_owait=True, with_issue=(c + LOOK < nch))

        for b in range(NBUF):
            out_wait((nout - 1) * NBUF + b, b)

    return k(granules, flat_idx)


def kernel(x, tables):
    xi = x.astype(jnp.int32)
    nf = F // 2
    outs = []
    for k in range(2):
        f0 = k * nf
        g = tables[f0:f0 + nf].transpose(0, 2, 1).reshape(nf * V, D)
        idx = xi[:, f0:f0 + nf].reshape(B * nf)
        outs.append(_sc_gather(g, idx, nf).reshape(B, nf * D))
    return jnp.concatenate(outs, axis=1)


# speedup vs baseline: 1.1112x; 1.1112x over previous
import functools

import jax
import jax.numpy as jnp
from jax import lax
from jax.experimental import pallas as pl
from jax.experimental.pallas import tpu as pltpu
from jax.experimental.pallas import tpu_sc as plsc

F = 26
V = 100000
D = 16
B = 16384
N = B * F
G16 = V // D

_INFO = plsc.get_sparse_core_info()
NC = _INFO.num_cores
NS = _INFO.num_subcores
NW = NC * NS
CH = 32
NBUF = 8
LOOK = 6


def _sc_gather(granules, flat_idx, nf):
    n = B * nf
    rpw = n // NW
    nch = rpw // CH
    nout = nch // NBUF
    assert rpw * NW == n and nch * CH == rpw and rpw % nf == 0
    assert nch % NBUF == 0 and nout >= 3 and NBUF == LOOK + 2
    mesh = plsc.VectorSubcoreMesh(core_axis_name="c", subcore_axis_name="s")

    @functools.partial(
        pl.kernel,
        mesh=mesh,
        out_type=jax.ShapeDtypeStruct((n, D), jnp.float32),
        scratch_types=[
            pltpu.VMEM((rpw,), jnp.int32),
            pltpu.VMEM((NBUF, D, CH), jnp.int32),
            pltpu.VMEM((NBUF, D, CH, D), jnp.float32),
            pltpu.VMEM((NBUF, CH, D), jnp.float32),
        ] + [pltpu.SemaphoreType.DMA] * (2 * NBUF),
        compiler_params=pltpu.CompilerParams(
            use_tc_tiling_on_sc=False, needs_layout_passes=False
        ),
    )
    def k(tab_hbm, idx_hbm, out_hbm, idx_v, pring, gbufs, obufs, *sems):
        gsems, osems = sems[:NBUF], sems[NBUF:]
        wid = lax.axis_index("s") * NC + lax.axis_index("c")
        base = wid * rpw
        pltpu.sync_copy(idx_hbm.at[pl.ds(base, rpw)], idx_v)

        lanes = lax.iota(jnp.int32, 16)

        def prep_chunk(c, b):
            def grp(gi, carry):
                s = gi * 16
                p = c * CH + s
                v = idx_v[pl.ds(p, 16)]
                m0 = ((p + lanes) % nf) * V + (v >> 4)
                for d in range(D):
                    pring.at[b, d][pl.ds(s, 16)] = m0 + d * G16
                return carry

            lax.fori_loop(0, CH // 16, grp, 0)

        def gather_start(b):
            for d in range(D):
                pltpu.async_copy(
                    tab_hbm.at[pring.at[b, d]], gbufs.at[b, d], gsems[b]
                )

        def gather_wait(b):
            for d in range(D):
                pltpu.make_async_copy(
                    tab_hbm.at[pring.at[b, d]], gbufs.at[b, d], gsems[b]
                ).wait()

        def extract_chunk(c, b):
            ob = obufs.at[b]

            def grp(gi, carry):
                s = gi * 16
                v = idx_v[pl.ds(c * CH + s, 16)]
                lo = v & 15
                rows = s + lanes
                for d in range(D):
                    vals = plsc.load_gather(gbufs.at[b, d], [rows, lo])
                    plsc.store_scatter(ob, [rows, lanes * 0 + d], vals)
                return carry

            lax.fori_loop(0, CH // 16, grp, 0)

        def out_start(c, b):
            pltpu.async_copy(
                obufs.at[b], out_hbm.at[pl.ds(base + c * CH, CH)], osems[b]
            )

        def out_wait(c, b):
            pltpu.make_async_copy(
                obufs.at[b], out_hbm.at[pl.ds(base + c * CH, CH)], osems[b]
            ).wait()

        def step(c, b, with_owait, with_issue):
            gather_wait(b)
            extract_chunk(c, b)
            out_start(c, b)
            if with_issue:
                j = c + LOOK
                bj = (b + LOOK) % NBUF
                prep_chunk(j, bj)
                if with_owait:
                    out_wait(j - NBUF, bj)
                gather_start(bj)

        for j in range(LOOK):
            prep_chunk(j, j % NBUF)
            gather_start(j % NBUF)

        for b in range(NBUF):
            step(b, b, with_owait=(b >= NBUF - LOOK), with_issue=True)

        def outer(cb, carry):
            for b in range(NBUF):
                step(cb * NBUF + b, b, with_owait=True, with_issue=True)
            return carry

        lax.fori_loop(1, nout - 1, outer, 0)

        for b in range(NBUF):
            c = (nout - 1) * NBUF + b
            step(c, b, with_owait=True, with_issue=(c + LOOK < nch))

        for b in range(NBUF):
            out_wait((nout - 1) * NBUF + b, b)

    return k(granules, flat_idx)


def kernel(x, tables):
    granules = tables.transpose(0, 2, 1).reshape(F * V, D)
    flat_idx = x.astype(jnp.int32).reshape(N)
    out = _sc_gather(granules, flat_idx, F)
    return out.reshape(B, F * D)
